# baseline (speedup 1.0000x reference)
import functools
import math

import jax
import jax.numpy as jnp
from jax import lax
from jax.experimental import pallas as pl
from jax.experimental.pallas import tpu as pltpu
from jax.experimental.pallas import tpu_sc as plsc

N = 10000
E = 320000
HEADS = 8
D_A = 24
D_V = 32
D_M = 64

NP = 10240
CS = 64
NCHUNKS = NP // CS
NTILES = 32
CPT = NCHUNKS // NTILES
CAPC = 16384
EBLK = 2000
GBLK = 16
QW = HEADS * D_A
VW = HEADS * D_V

_MESH = plsc.VectorSubcoreMesh(core_axis_name="c", subcore_axis_name="s")


def _wid():
    return lax.axis_index("s") * 2 + lax.axis_index("c")


@functools.partial(
    pl.kernel,
    out_type=(
        jax.ShapeDtypeStruct((NCHUNKS * CAPC,), jnp.int32),
        jax.ShapeDtypeStruct((NCHUNKS * 16,), jnp.int32),
    ),
    mesh=_MESH,
    scratch_types=[
        pltpu.VMEM((EBLK,), jnp.int32),
        pltpu.VMEM((EBLK,), jnp.int32),
        pltpu.VMEM((CPT * (CAPC + 16),), jnp.int32),
        pltpu.VMEM((16,), jnp.int32),
    ],
    compiler_params=pltpu.CompilerParams(needs_layout_passes=False),
)
def _bucket_kernel(src_hbm, dst_hbm, buckets, counts, sblk, dblk, lists, cvec):
    wid = _wid()
    base_node = wid * (CPT * CS)

    def blk_body(b, cur):
        pltpu.sync_copy(src_hbm.at[pl.ds(b * EBLK, EBLK)], sblk)
        pltpu.sync_copy(dst_hbm.at[pl.ds(b * EBLK, EBLK)], dblk)

        def vec_body(j, cur):
            sv = sblk[pl.ds(j * 16, 16)]
            dv = dblk[pl.ds(j * 16, 16)]
            dlt = dv - base_node
            new = []
            for cc in range(CPT):
                lo = cc * CS
                mask = (dlt >= lo) & (dlt < lo + CS)
                packed = (sv << 6) | (dlt - lo)
                mi = jnp.where(mask, jnp.full((16,), 1, jnp.int32),
                               jnp.full((16,), 0, jnp.int32))
                incl = plsc.cumsum(mi)
                base_pos = jnp.full((16,), cc * (CAPC + 16) + cur[cc],
                                    jnp.int32)
                pos = base_pos + (incl - mi)
                plsc.store_scatter(lists, [pos], packed, mask=mask)
                new.append(cur[cc] + incl[15])
            return tuple(new)

        return lax.fori_loop(0, EBLK // 16, vec_body, cur)

    cur = lax.fori_loop(0, E // EBLK, blk_body,
                        tuple(jnp.int32(0) for _ in range(CPT)))

    for cc in range(CPT):
        c = wid * CPT + cc
        cvec[...] = jnp.full((16,), cur[cc], jnp.int32)
        pltpu.sync_copy(cvec, counts.at[pl.ds(c * 16, 16)])
        nb = (cur[cc] + 2047) // 2048

        def wr_body(bb, _, cc=cc, c=c):
            pltpu.sync_copy(
                lists.at[pl.ds(cc * (CAPC + 16) + bb * 2048, 2048)],
                buckets.at[pl.ds(c * CAPC + bb * 2048, 2048)])
            return 0

        lax.fori_loop(0, nb, wr_body, 0)


def _make_edge_kernel(d_in):
    roww = QW + VW + D_M + d_in
    voff = QW
    moff = QW + VW
    xoff = QW + VW + D_M
    nxv = d_in // 16
    inv_sqrt = 1.0 / math.sqrt(float(D_A))

    @functools.partial(
        pl.kernel,
        out_type=(
            jax.ShapeDtypeStruct((NP, 16), jnp.float32),
            jax.ShapeDtypeStruct((NP, VW), jnp.float32),
            jax.ShapeDtypeStruct((NP, D_M), jnp.float32),
            jax.ShapeDtypeStruct((NP, d_in), jnp.float32),
            jax.ShapeDtypeStruct((NP, 16), jnp.float32),
        ),
        mesh=_MESH,
        scratch_types=[
            pltpu.VMEM((CS, QW), jnp.float32),
            pltpu.VMEM((CS, 16), jnp.float32),
            pltpu.VMEM((CS, VW), jnp.float32),
            pltpu.VMEM((CS, D_M), jnp.float32),
            pltpu.VMEM((CS, d_in), jnp.float32),
            pltpu.VMEM((CS, 16), jnp.float32),
            pltpu.VMEM((CS, 16), jnp.float32),
            pltpu.VMEM((CS, VW), jnp.float32),
            pltpu.VMEM((CS, D_M), jnp.float32),
            pltpu.VMEM((CS, 16), jnp.float32),
            pltpu.VMEM((GBLK,), jnp.int32),
            pltpu.VMEM((GBLK,), jnp.int32),
            pltpu.VMEM((GBLK, roww), jnp.float32),
            pltpu.VMEM((16,), jnp.int32),
            pltpu.SemaphoreType.DMA,
        ],
        compiler_params=pltpu.CompilerParams(needs_layout_passes=False),
    )
    def edge_kernel(q_hbm, r_hbm, buckets, counts,
                    denom, aggv, maxm, sumx, cnt,
                    qv, accD, accA, accM, accX, accC,
                    accD2, accA2, accM2, accC2,
                    blk, idx, rows, cvec, sem):
        wid = _wid()
        zero16 = jnp.zeros((16,), jnp.float32)
        neg = jnp.full((16,), -3.0e38, jnp.float32)
        one16 = jnp.full((16,), 1.0, jnp.float32)

        def chunk_body(cc, _):
            c = wid * CPT + cc
            base = c * CS
            pltpu.sync_copy(counts.at[pl.ds(c * 16, 16)], cvec)
            ctotal = cvec[...][0]
            pltpu.sync_copy(q_hbm.at[pl.ds(base, CS)], qv)

            def init_body(i, _):
                accD[i, :] = zero16
                accC[i, :] = zero16
                accD2[i, :] = zero16
                accC2[i, :] = zero16
                for j in range(VW // 16):
                    accA[i, pl.ds(j * 16, 16)] = zero16
                    accA2[i, pl.ds(j * 16, 16)] = zero16
                for j in range(D_M // 16):
                    accM[i, pl.ds(j * 16, 16)] = neg
                    accM2[i, pl.ds(j * 16, 16)] = neg
                for j in range(nxv):
                    accX[i, pl.ds(j * 16, 16)] = zero16
                return 0

            lax.fori_loop(0, CS, init_body, 0)

            nb = (ctotal + GBLK - 1) // GBLK

            def blk_body(b, _, c=c, ctotal=ctotal):
                pltpu.sync_copy(buckets.at[pl.ds(c * CAPC + b * GBLK, GBLK)], blk)
                for jj in range(GBLK // 16):
                    sl = pl.ds(jj * 16, 16)
                    idx[sl] = jnp.clip(lax.shift_right_logical(blk[sl], 6),
                                       0, NP - 1)
                pltpu.async_copy(r_hbm.at[idx], rows, sem).wait()

                def do_edge(e, dl, aD, aA, aM, aC):
                    acc0 = rows[e, pl.ds(0, 16)] * qv[dl, pl.ds(0, 16)]
                    acc1 = rows[e, pl.ds(16, 16)] * qv[dl, pl.ds(16, 16)]
                    acc2 = rows[e, pl.ds(32, 16)] * qv[dl, pl.ds(32, 16)]
                    for j in range(3, QW // 16, 3):
                        acc0 = acc0 + rows[e, pl.ds(j * 16, 16)] * qv[dl, pl.ds(j * 16, 16)]
                        acc1 = acc1 + rows[e, pl.ds(j * 16 + 16, 16)] * qv[dl, pl.ds(j * 16 + 16, 16)]
                        acc2 = acc2 + rows[e, pl.ds(j * 16 + 32, 16)] * qv[dl, pl.ds(j * 16 + 32, 16)]
                    acc = acc0 + acc1 + acc2
                    exv = jnp.exp((acc + lax.rev(acc, (0,))) * inv_sqrt)
                    aD[dl, :] = aD[dl, :] + exv
                    aC[dl, :] = aC[dl, :] + one16
                    for j in range(VW // 16):
                        sl = pl.ds(j * 16, 16)
                        w = jnp.full((16,), exv[j // 2], jnp.float32)
                        aA[dl, sl] = (aA[dl, sl]
                                      + rows[e, pl.ds(voff + j * 16, 16)] * w)
                    for j in range(D_M // 16):
                        sl = pl.ds(j * 16, 16)
                        aM[dl, sl] = jnp.maximum(
                            aM[dl, sl], rows[e, pl.ds(moff + j * 16, 16)])
                    for j in range(nxv):
                        sl = pl.ds(j * 16, 16)
                        accX[dl, sl] = (accX[dl, sl]
                                        + rows[e, pl.ds(xoff + j * 16, 16)])

                banks = ((accD, accA, accM, accC), (accD2, accA2, accM2, accC2))

                def grp_body(jj, _):
                    ev = blk[pl.ds(jj * 16, 16)]
                    dlv = ev & 63
                    grp_end = b * GBLK + jj * 16 + 16

                    @pl.when(grp_end <= ctotal)
                    def _():
                        for e2 in range(16):
                            do_edge(jj * 16 + e2, dlv[e2], *banks[e2 % 2])

                    @pl.when(grp_end > ctotal)
                    def _():
                        for e2 in range(16):
                            e = jj * 16 + e2

                            @pl.when(b * GBLK + e < ctotal)
                            def _(e=e, e2=e2):
                                do_edge(e, dlv[e2], *banks[e2 % 2])
                    return 0

                lax.fori_loop(0, GBLK // 16, grp_body, 0)
                return 0

            lax.fori_loop(0, nb, blk_body, 0)

            def merge_body(i, _):
                accD[i, :] = accD[i, :] + accD2[i, :]
                accC[i, :] = accC[i, :] + accC2[i, :]
                for j in range(VW // 16):
                    sl = pl.ds(j * 16, 16)
                    accA[i, sl] = accA[i, sl] + accA2[i, sl]
                for j in range(D_M // 16):
                    sl = pl.ds(j * 16, 16)
                    accM[i, sl] = jnp.maximum(accM[i, sl], accM2[i, sl])
                return 0

            lax.fori_loop(0, CS, merge_body, 0)

            pltpu.sync_copy(accD, denom.at[pl.ds(base, CS)])
            pltpu.sync_copy(accA, aggv.at[pl.ds(base, CS)])
            pltpu.sync_copy(accM, maxm.at[pl.ds(base, CS)])
            pltpu.sync_copy(accX, sumx.at[pl.ds(base, CS)])
            pltpu.sync_copy(accC, cnt.at[pl.ds(base, CS)])
            return 0

        lax.fori_loop(0, CPT, chunk_body, 0)

    return edge_kernel


_edge_kernel_128 = _make_edge_kernel(128)
_edge_kernel_256 = _make_edge_kernel(256)


def _proj_body(x_ref, wq_ref, wk_ref, wv_ref, wm_ref, q_ref, r_ref):
    xb = x_ref[...]
    q_ref[...] = jnp.dot(xb, wq_ref[...], preferred_element_type=jnp.float32)
    r_ref[:, 0:QW] = jnp.dot(xb, wk_ref[...],
                             preferred_element_type=jnp.float32)
    r_ref[:, QW:QW + VW] = jnp.dot(xb, wv_ref[...],
                                   preferred_element_type=jnp.float32)
    r_ref[:, QW + VW:QW + VW + D_M] = jnp.dot(
        xb, wm_ref[...], preferred_element_type=jnp.float32)
    r_ref[:, QW + VW + D_M:] = xb


def _proj(x, wq, wk, wv, wm):
    d = x.shape[1]
    roww = QW + VW + D_M + d
    br = 1024
    grid = (NP // br,)
    return pl.pallas_call(
        _proj_body,
        grid=grid,
        in_specs=[
            pl.BlockSpec((br, d), lambda i: (i, 0)),
            pl.BlockSpec((d, QW), lambda i: (0, 0)),
            pl.BlockSpec((d, QW), lambda i: (0, 0)),
            pl.BlockSpec((d, VW), lambda i: (0, 0)),
            pl.BlockSpec((d, D_M), lambda i: (0, 0)),
        ],
        out_specs=[
            pl.BlockSpec((br, QW), lambda i: (i, 0)),
            pl.BlockSpec((br, roww), lambda i: (i, 0)),
        ],
        out_shape=[
            jax.ShapeDtypeStruct((NP, QW), jnp.float32),
            jax.ShapeDtypeStruct((NP, roww), jnp.float32),
        ],
    )(x, wq, wk, wv, wm)


def _make_finish_body(final):
    def body(x_ref, denom_ref, aggv_ref, maxm_ref, sumx_ref, cnt_ref,
             wgx_ref, wgm_ref, wgz_ref, wox_ref, wog_ref, o_ref):
        xb = x_ref[...]
        cnt1 = cnt_ref[:, 0:1]
        has = cnt1 > 0.0
        mean = sumx_ref[...] / jnp.maximum(cnt1, 1.0)
        mm = jnp.where(has, maxm_ref[...], 0.0)
        g = jax.nn.sigmoid(
            jnp.dot(xb, wgx_ref[...], preferred_element_type=jnp.float32)
            + jnp.dot(mm, wgm_ref[...], preferred_element_type=jnp.float32)
            + jnp.dot(mean, wgz_ref[...], preferred_element_type=jnp.float32))
        expand = jnp.repeat(jnp.eye(HEADS, dtype=jnp.float32), D_V, axis=1)
        dnex = jnp.dot(denom_ref[:, 0:HEADS], expand,
                       preferred_element_type=jnp.float32)
        gex = jnp.dot(g, expand, preferred_element_type=jnp.float32)
        gated = gex * aggv_ref[...] / (dnex + 1e-16)
        out = (jnp.dot(xb, wox_ref[...], preferred_element_type=jnp.float32)
               + jnp.dot(gated, wog_ref[...],
                         preferred_element_type=jnp.float32))
        out = jnp.where(out >= 0.0, out, 0.1 * out)
        if final:
            m = jnp.max(out, axis=1, keepdims=True)
            out = out - m - jnp.log(
                jnp.sum(jnp.exp(out - m), axis=1, keepdims=True))
        o_ref[...] = out

    return body


def _finish(x, denom, aggv, maxm, sumx, cnt, wg, wo, final):
    d = x.shape[1]
    do = wo.shape[1]
    wgx = wg[:d]
    wgm = wg[d:d + D_M]
    wgz = wg[d + D_M:]
    wox = wo[:d]
    wog = wo[d:]
    br = 1024
    grid = (NP // br,)
    return pl.pallas_call(
        _make_finish_body(final),
        grid=grid,
        in_specs=[
            pl.BlockSpec((br, d), lambda i: (i, 0)),
            pl.BlockSpec((br, 16), lambda i: (i, 0)),
            pl.BlockSpec((br, VW), lambda i: (i, 0)),
            pl.BlockSpec((br, D_M), lambda i: (i, 0)),
            pl.BlockSpec((br, d), lambda i: (i, 0)),
            pl.BlockSpec((br, 16), lambda i: (i, 0)),
            pl.BlockSpec((d, HEADS), lambda i: (0, 0)),
            pl.BlockSpec((D_M, HEADS), lambda i: (0, 0)),
            pl.BlockSpec((d, HEADS), lambda i: (0, 0)),
            pl.BlockSpec((d, do), lambda i: (0, 0)),
            pl.BlockSpec((VW, do), lambda i: (0, 0)),
        ],
        out_specs=pl.BlockSpec((br, do), lambda i: (i, 0)),
        out_shape=jax.ShapeDtypeStruct((NP, do), jnp.float32),
    )(x, denom, aggv, maxm, sumx, cnt, wgx, wgm, wgz, wox, wog)


def _layer(x, buckets, counts, Wq, Wk, Wv, Wm, Wg, Wo, edge_kernel, final):
    d = x.shape[1]
    perm = []
    for j in range(QW // 16):
        for lane in range(16):
            h = lane if lane < 8 else 15 - lane
            a = 2 * j if lane < 8 else 2 * j + 1
            perm.append(h * D_A + a)
    perm = jnp.asarray(perm, jnp.int32)
    wq_p = Wq[:, perm]
    wk_p = Wk[:, perm]
    q, r = _proj(x, wq_p, wk_p, Wv, Wm)
    denom, aggv, maxm, sumx, cnt = edge_kernel(q, r, buckets, counts)
    return _finish(x, denom, aggv, maxm, sumx, cnt, Wg, Wo, final)


def kernel(x, edge_index, Wq1, Wk1, Wv1, Wm1, Wg1, Wo1,
           Wq2, Wk2, Wv2, Wm2, Wg2, Wo2):
    src = edge_index[0]
    dst = edge_index[1]
    buckets, counts = _bucket_kernel(src, dst)
    xp = jnp.pad(x, ((0, NP - N), (0, 0)))
    h = _layer(xp, buckets, counts, Wq1, Wk1, Wv1, Wm1, Wg1, Wo1,
               _edge_kernel_128, final=False)
    out = _layer(h, buckets, counts, Wq2, Wk2, Wv2, Wm2, Wg2, Wo2,
                 _edge_kernel_256, final=True)
    return out[:N]


# speedup vs baseline: 1.1636x; 1.1636x over previous
import functools
import math

import jax
import jax.numpy as jnp
from jax import lax
from jax.experimental import pallas as pl
from jax.experimental.pallas import tpu as pltpu
from jax.experimental.pallas import tpu_sc as plsc

N = 10000
E = 320000
HEADS = 8
D_A = 24
D_V = 32
D_M = 64

NP = 10240
CS = 64
NCHUNKS = NP // CS
NTILES = 32
CPT = NCHUNKS // NTILES
CAPC = 16384
EBLK = 2000
GBLK = 32
QW = HEADS * D_A
VW = HEADS * D_V

_MESH = plsc.VectorSubcoreMesh(core_axis_name="c", subcore_axis_name="s")


def _wid():
    return lax.axis_index("s") * 2 + lax.axis_index("c")


@functools.partial(
    pl.kernel,
    out_type=(
        jax.ShapeDtypeStruct((NCHUNKS * CAPC,), jnp.int32),
        jax.ShapeDtypeStruct((NCHUNKS * 16,), jnp.int32),
    ),
    mesh=_MESH,
    scratch_types=[
        pltpu.VMEM((EBLK,), jnp.int32),
        pltpu.VMEM((EBLK,), jnp.int32),
        pltpu.VMEM((CPT * (CAPC + 16),), jnp.int32),
        pltpu.VMEM((16,), jnp.int32),
    ],
    compiler_params=pltpu.CompilerParams(needs_layout_passes=False),
)
def _bucket_kernel(src_hbm, dst_hbm, buckets, counts, sblk, dblk, lists, cvec):
    wid = _wid()
    base_node = wid * (CPT * CS)

    def blk_body(b, cur):
        pltpu.sync_copy(src_hbm.at[pl.ds(b * EBLK, EBLK)], sblk)
        pltpu.sync_copy(dst_hbm.at[pl.ds(b * EBLK, EBLK)], dblk)

        def vec_body(j, cur):
            sv = sblk[pl.ds(j * 16, 16)]
            dv = dblk[pl.ds(j * 16, 16)]
            dlt = dv - base_node
            new = []
            for cc in range(CPT):
                lo = cc * CS
                mask = (dlt >= lo) & (dlt < lo + CS)
                packed = (sv << 6) | (dlt - lo)
                mi = jnp.where(mask, jnp.full((16,), 1, jnp.int32),
                               jnp.full((16,), 0, jnp.int32))
                incl = plsc.cumsum(mi)
                base_pos = jnp.full((16,), cc * (CAPC + 16) + cur[cc],
                                    jnp.int32)
                pos = base_pos + (incl - mi)
                plsc.store_scatter(lists, [pos], packed, mask=mask)
                new.append(cur[cc] + incl[15])
            return tuple(new)

        return lax.fori_loop(0, EBLK // 16, vec_body, cur)

    cur = lax.fori_loop(0, E // EBLK, blk_body,
                        tuple(jnp.int32(0) for _ in range(CPT)))

    for cc in range(CPT):
        c = wid * CPT + cc
        cvec[...] = jnp.full((16,), cur[cc], jnp.int32)
        pltpu.sync_copy(cvec, counts.at[pl.ds(c * 16, 16)])
        nb = (cur[cc] + 2047) // 2048

        def wr_body(bb, _, cc=cc, c=c):
            pltpu.sync_copy(
                lists.at[pl.ds(cc * (CAPC + 16) + bb * 2048, 2048)],
                buckets.at[pl.ds(c * CAPC + bb * 2048, 2048)])
            return 0

        lax.fori_loop(0, nb, wr_body, 0)


def _make_edge_kernel(d_in):
    roww = QW + VW + D_M + d_in
    voff = QW
    moff = QW + VW
    xoff = QW + VW + D_M
    nxv = d_in // 16
    inv_sqrt = 1.0 / math.sqrt(float(D_A))

    @functools.partial(
        pl.kernel,
        out_type=(
            jax.ShapeDtypeStruct((NP, 16), jnp.float32),
            jax.ShapeDtypeStruct((NP, VW), jnp.float32),
            jax.ShapeDtypeStruct((NP, D_M), jnp.float32),
            jax.ShapeDtypeStruct((NP, d_in), jnp.float32),
            jax.ShapeDtypeStruct((NP, 16), jnp.float32),
        ),
        mesh=_MESH,
        scratch_types=[
            pltpu.VMEM((CS, QW), jnp.float32),
            pltpu.VMEM((CS, 16), jnp.float32),
            pltpu.VMEM((CS, VW), jnp.float32),
            pltpu.VMEM((CS, D_M), jnp.float32),
            pltpu.VMEM((CS, d_in), jnp.float32),
            pltpu.VMEM((CS, 16), jnp.float32),
            pltpu.VMEM((2 * GBLK,), jnp.int32),
            pltpu.VMEM((GBLK,), jnp.int32),
            pltpu.VMEM((GBLK,), jnp.int32),
            pltpu.VMEM((2 * GBLK, roww), jnp.float32),
            pltpu.VMEM((16,), jnp.int32),
            pltpu.SemaphoreType.DMA,
            pltpu.SemaphoreType.DMA,
        ],
        compiler_params=pltpu.CompilerParams(needs_layout_passes=False),
    )
    def edge_kernel(q_hbm, r_hbm, buckets, counts,
                    denom, aggv, maxm, sumx, cnt,
                    qv, accD, accA, accM, accX, accC,
                    blk, idx0, idx1, rows, cvec, sem0, sem1):
        wid = _wid()
        zero16 = jnp.zeros((16,), jnp.float32)
        neg = jnp.full((16,), -3.0e38, jnp.float32)
        one16 = jnp.full((16,), 1.0, jnp.float32)

        def chunk_body(cc, _):
            c = wid * CPT + cc
            base = c * CS
            pltpu.sync_copy(counts.at[pl.ds(c * 16, 16)], cvec)
            ctotal = cvec[...][0]
            pltpu.sync_copy(q_hbm.at[pl.ds(base, CS)], qv)

            def init_body(i, _):
                accD[i, :] = zero16
                accC[i, :] = zero16
                for j in range(VW // 16):
                    accA[i, pl.ds(j * 16, 16)] = zero16
                for j in range(D_M // 16):
                    accM[i, pl.ds(j * 16, 16)] = neg
                for j in range(nxv):
                    accX[i, pl.ds(j * 16, 16)] = zero16
                return 0

            lax.fori_loop(0, CS, init_body, 0)

            nfull = ctotal // GBLK
            tail = ctotal - nfull * GBLK

            def fire(b, c=c):
                par = lax.rem(b, 2)
                hb = par * GBLK

                @pl.when(par == 0)
                def _():
                    pltpu.sync_copy(
                        buckets.at[pl.ds(c * CAPC + b * GBLK, GBLK)],
                        blk.at[pl.ds(0, GBLK)])
                    for jj in range(GBLK // 16):
                        sl = pl.ds(jj * 16, 16)
                        idx0[sl] = jnp.clip(
                            lax.shift_right_logical(blk[sl], 6), 0, NP - 1)
                    pltpu.async_copy(r_hbm.at[idx0],
                                     rows.at[pl.ds(0, GBLK)], sem0)

                @pl.when(par == 1)
                def _():
                    pltpu.sync_copy(
                        buckets.at[pl.ds(c * CAPC + b * GBLK, GBLK)],
                        blk.at[pl.ds(GBLK, GBLK)])
                    for jj in range(GBLK // 16):
                        idx1[pl.ds(jj * 16, 16)] = jnp.clip(
                            lax.shift_right_logical(
                                blk[pl.ds(GBLK + jj * 16, 16)], 6),
                            0, NP - 1)
                    pltpu.async_copy(r_hbm.at[idx1],
                                     rows.at[pl.ds(GBLK, GBLK)], sem1)
                del hb

            def drain(b):
                par = lax.rem(b, 2)

                @pl.when(par == 0)
                def _():
                    pltpu.make_async_copy(r_hbm.at[idx0],
                                          rows.at[pl.ds(0, GBLK)],
                                          sem0).wait()

                @pl.when(par == 1)
                def _():
                    pltpu.make_async_copy(r_hbm.at[idx1],
                                          rows.at[pl.ds(GBLK, GBLK)],
                                          sem1).wait()

            def do_edge(e, dl):
                acc0 = rows[e, pl.ds(0, 16)] * qv[dl, pl.ds(0, 16)]
                acc1 = rows[e, pl.ds(16, 16)] * qv[dl, pl.ds(16, 16)]
                acc2 = rows[e, pl.ds(32, 16)] * qv[dl, pl.ds(32, 16)]
                for j in range(3, QW // 16, 3):
                    acc0 = acc0 + rows[e, pl.ds(j * 16, 16)] * qv[dl, pl.ds(j * 16, 16)]
                    acc1 = acc1 + rows[e, pl.ds(j * 16 + 16, 16)] * qv[dl, pl.ds(j * 16 + 16, 16)]
                    acc2 = acc2 + rows[e, pl.ds(j * 16 + 32, 16)] * qv[dl, pl.ds(j * 16 + 32, 16)]
                acc = acc0 + acc1 + acc2
                exv = jnp.exp((acc + lax.rev(acc, (0,))) * inv_sqrt)
                accD[dl, :] = accD[dl, :] + exv
                accC[dl, :] = accC[dl, :] + one16
                for j in range(VW // 16):
                    sl = pl.ds(j * 16, 16)
                    w = jnp.full((16,), exv[j // 2], jnp.float32)
                    accA[dl, sl] = (accA[dl, sl]
                                    + rows[e, pl.ds(voff + j * 16, 16)] * w)
                for j in range(D_M // 16):
                    sl = pl.ds(j * 16, 16)
                    accM[dl, sl] = jnp.maximum(
                        accM[dl, sl], rows[e, pl.ds(moff + j * 16, 16)])
                for j in range(nxv):
                    sl = pl.ds(j * 16, 16)
                    accX[dl, sl] = (accX[dl, sl]
                                    + rows[e, pl.ds(xoff + j * 16, 16)])

            @pl.when(nfull > 0)
            def _():
                fire(jnp.int32(0))

            def blk_body(b, _):
                @pl.when(b + 1 < nfull)
                def _():
                    fire(b + 1)

                drain(b)
                hb = lax.rem(b, 2) * GBLK

                def grp_body(jj, _):
                    s0 = hb + jj * 16
                    ev = blk[pl.ds(s0, 16)]
                    dlv = ev & 63
                    for e2 in range(16):
                        do_edge(s0 + e2, dlv[e2])
                    return 0

                lax.fori_loop(0, GBLK // 16, grp_body, 0)
                return 0

            lax.fori_loop(0, nfull, blk_body, 0)

            @pl.when(tail > 0)
            def _():
                pltpu.sync_copy(
                    buckets.at[pl.ds(c * CAPC + nfull * GBLK, GBLK)],
                    blk.at[pl.ds(0, GBLK)])
                for jj in range(GBLK // 16):
                    sl = pl.ds(jj * 16, 16)
                    idx0[sl] = jnp.clip(
                        lax.shift_right_logical(blk[sl], 6), 0, NP - 1)
                pltpu.async_copy(r_hbm.at[idx0],
                                 rows.at[pl.ds(0, GBLK)], sem0).wait()

                def tgrp_body(jj, _):
                    ev = blk[pl.ds(jj * 16, 16)]
                    dlv = ev & 63
                    for e2 in range(16):
                        e = jj * 16 + e2

                        @pl.when(e < tail)
                        def _(e=e, e2=e2):
                            do_edge(e, dlv[e2])
                    return 0

                lax.fori_loop(0, GBLK // 16, tgrp_body, 0)

            pltpu.sync_copy(accD, denom.at[pl.ds(base, CS)])
            pltpu.sync_copy(accA, aggv.at[pl.ds(base, CS)])
            pltpu.sync_copy(accM, maxm.at[pl.ds(base, CS)])
            pltpu.sync_copy(accX, sumx.at[pl.ds(base, CS)])
            pltpu.sync_copy(accC, cnt.at[pl.ds(base, CS)])
            return 0

        lax.fori_loop(0, CPT, chunk_body, 0)

    return edge_kernel


_edge_kernel_128 = _make_edge_kernel(128)
_edge_kernel_256 = _make_edge_kernel(256)


def _proj_body(x_ref, wq_ref, wk_ref, wv_ref, wm_ref, q_ref, r_ref):
    xb = x_ref[...]
    q_ref[...] = jnp.dot(xb, wq_ref[...], preferred_element_type=jnp.float32)
    r_ref[:, 0:QW] = jnp.dot(xb, wk_ref[...],
                             preferred_element_type=jnp.float32)
    r_ref[:, QW:QW + VW] = jnp.dot(xb, wv_ref[...],
                                   preferred_element_type=jnp.float32)
    r_ref[:, QW + VW:QW + VW + D_M] = jnp.dot(
        xb, wm_ref[...], preferred_element_type=jnp.float32)
    r_ref[:, QW + VW + D_M:] = xb


def _proj(x, wq, wk, wv, wm):
    d = x.shape[1]
    roww = QW + VW + D_M + d
    br = 1024
    grid = (NP // br,)
    return pl.pallas_call(
        _proj_body,
        grid=grid,
        in_specs=[
            pl.BlockSpec((br, d), lambda i: (i, 0)),
            pl.BlockSpec((d, QW), lambda i: (0, 0)),
            pl.BlockSpec((d, QW), lambda i: (0, 0)),
            pl.BlockSpec((d, VW), lambda i: (0, 0)),
            pl.BlockSpec((d, D_M), lambda i: (0, 0)),
        ],
        out_specs=[
            pl.BlockSpec((br, QW), lambda i: (i, 0)),
            pl.BlockSpec((br, roww), lambda i: (i, 0)),
        ],
        out_shape=[
            jax.ShapeDtypeStruct((NP, QW), jnp.float32),
            jax.ShapeDtypeStruct((NP, roww), jnp.float32),
        ],
    )(x, wq, wk, wv, wm)


def _make_finish_body(final):
    def body(x_ref, denom_ref, aggv_ref, maxm_ref, sumx_ref, cnt_ref,
             wgx_ref, wgm_ref, wgz_ref, wox_ref, wog_ref, o_ref):
        xb = x_ref[...]
        cnt1 = cnt_ref[:, 0:1]
        has = cnt1 > 0.0
        mean = sumx_ref[...] / jnp.maximum(cnt1, 1.0)
        mm = jnp.where(has, maxm_ref[...], 0.0)
        g = jax.nn.sigmoid(
            jnp.dot(xb, wgx_ref[...], preferred_element_type=jnp.float32)
            + jnp.dot(mm, wgm_ref[...], preferred_element_type=jnp.float32)
            + jnp.dot(mean, wgz_ref[...], preferred_element_type=jnp.float32))
        expand = jnp.repeat(jnp.eye(HEADS, dtype=jnp.float32), D_V, axis=1)
        dnex = jnp.dot(denom_ref[:, 0:HEADS], expand,
                       preferred_element_type=jnp.float32)
        gex = jnp.dot(g, expand, preferred_element_type=jnp.float32)
        gated = gex * aggv_ref[...] / (dnex + 1e-16)
        out = (jnp.dot(xb, wox_ref[...], preferred_element_type=jnp.float32)
               + jnp.dot(gated, wog_ref[...],
                         preferred_element_type=jnp.float32))
        out = jnp.where(out >= 0.0, out, 0.1 * out)
        if final:
            m = jnp.max(out, axis=1, keepdims=True)
            out = out - m - jnp.log(
                jnp.sum(jnp.exp(out - m), axis=1, keepdims=True))
        o_ref[...] = out

    return body


def _finish(x, denom, aggv, maxm, sumx, cnt, wg, wo, final):
    d = x.shape[1]
    do = wo.shape[1]
    wgx = wg[:d]
    wgm = wg[d:d + D_M]
    wgz = wg[d + D_M:]
    wox = wo[:d]
    wog = wo[d:]
    br = 1024
    grid = (NP // br,)
    return pl.pallas_call(
        _make_finish_body(final),
        grid=grid,
        in_specs=[
            pl.BlockSpec((br, d), lambda i: (i, 0)),
            pl.BlockSpec((br, 16), lambda i: (i, 0)),
            pl.BlockSpec((br, VW), lambda i: (i, 0)),
            pl.BlockSpec((br, D_M), lambda i: (i, 0)),
            pl.BlockSpec((br, d), lambda i: (i, 0)),
            pl.BlockSpec((br, 16), lambda i: (i, 0)),
            pl.BlockSpec((d, HEADS), lambda i: (0, 0)),
            pl.BlockSpec((D_M, HEADS), lambda i: (0, 0)),
            pl.BlockSpec((d, HEADS), lambda i: (0, 0)),
            pl.BlockSpec((d, do), lambda i: (0, 0)),
            pl.BlockSpec((VW, do), lambda i: (0, 0)),
        ],
        out_specs=pl.BlockSpec((br, do), lambda i: (i, 0)),
        out_shape=jax.ShapeDtypeStruct((NP, do), jnp.float32),
    )(x, denom, aggv, maxm, sumx, cnt, wgx, wgm, wgz, wox, wog)


def _layer(x, buckets, counts, Wq, Wk, Wv, Wm, Wg, Wo, edge_kernel, final):
    d = x.shape[1]
    perm = []
    for j in range(QW // 16):
        for lane in range(16):
            h = lane if lane < 8 else 15 - lane
            a = 2 * j if lane < 8 else 2 * j + 1
            perm.append(h * D_A + a)
    perm = jnp.asarray(perm, jnp.int32)
    wq_p = Wq[:, perm]
    wk_p = Wk[:, perm]
    q, r = _proj(x, wq_p, wk_p, Wv, Wm)
    denom, aggv, maxm, sumx, cnt = edge_kernel(q, r, buckets, counts)
    return _finish(x, denom, aggv, maxm, sumx, cnt, Wg, Wo, final)


def kernel(x, edge_index, Wq1, Wk1, Wv1, Wm1, Wg1, Wo1,
           Wq2, Wk2, Wv2, Wm2, Wg2, Wo2):
    src = edge_index[0]
    dst = edge_index[1]
    buckets, counts = _bucket_kernel(src, dst)
    xp = jnp.pad(x, ((0, NP - N), (0, 0)))
    h = _layer(xp, buckets, counts, Wq1, Wk1, Wv1, Wm1, Wg1, Wo1,
               _edge_kernel_128, final=False)
    out = _layer(h, buckets, counts, Wq2, Wk2, Wv2, Wm2, Wg2, Wo2,
                 _edge_kernel_256, final=True)
    return out[:N]


# speedup vs baseline: 1.1882x; 1.0211x over previous
import functools
import math

import jax
import jax.numpy as jnp
from jax import lax
from jax.experimental import pallas as pl
from jax.experimental.pallas import tpu as pltpu
from jax.experimental.pallas import tpu_sc as plsc

N = 10000
E = 320000
HEADS = 8
D_A = 24
D_V = 32
D_M = 64

NP = 10240
CS = 64
NCHUNKS = NP // CS
NTILES = 32
CPT = NCHUNKS // NTILES
CAPC = 16384
EBLK = 2000
GBLK = 16
QW = HEADS * D_A
VW = HEADS * D_V

_MESH = plsc.VectorSubcoreMesh(core_axis_name="c", subcore_axis_name="s")


def _wid():
    return lax.axis_index("s") * 2 + lax.axis_index("c")


@functools.partial(
    pl.kernel,
    out_type=(
        jax.ShapeDtypeStruct((NCHUNKS * CAPC,), jnp.int32),
        jax.ShapeDtypeStruct((NCHUNKS * 16,), jnp.int32),
    ),
    mesh=_MESH,
    scratch_types=[
        pltpu.VMEM((EBLK,), jnp.int32),
        pltpu.VMEM((EBLK,), jnp.int32),
        pltpu.VMEM((CPT * (CAPC + 16),), jnp.int32),
        pltpu.VMEM((16,), jnp.int32),
    ],
    compiler_params=pltpu.CompilerParams(needs_layout_passes=False),
)
def _bucket_kernel(src_hbm, dst_hbm, buckets, counts, sblk, dblk, lists, cvec):
    wid = _wid()
    base_node = wid * (CPT * CS)

    def blk_body(b, cur):
        pltpu.sync_copy(src_hbm.at[pl.ds(b * EBLK, EBLK)], sblk)
        pltpu.sync_copy(dst_hbm.at[pl.ds(b * EBLK, EBLK)], dblk)

        def vec_body(j, cur):
            sv = sblk[pl.ds(j * 16, 16)]
            dv = dblk[pl.ds(j * 16, 16)]
            dlt = dv - base_node
            new = []
            for cc in range(CPT):
                lo = cc * CS
                mask = (dlt >= lo) & (dlt < lo + CS)
                packed = (sv << 6) | (dlt - lo)
                mi = jnp.where(mask, jnp.full((16,), 1, jnp.int32),
                               jnp.full((16,), 0, jnp.int32))
                incl = plsc.cumsum(mi)
                base_pos = jnp.full((16,), cc * (CAPC + 16) + cur[cc],
                                    jnp.int32)
                pos = base_pos + (incl - mi)
                plsc.store_scatter(lists, [pos], packed, mask=mask)
                new.append(cur[cc] + incl[15])
            return tuple(new)

        return lax.fori_loop(0, EBLK // 16, vec_body, cur)

    cur = lax.fori_loop(0, E // EBLK, blk_body,
                        tuple(jnp.int32(0) for _ in range(CPT)))

    for cc in range(CPT):
        c = wid * CPT + cc
        cvec[...] = jnp.full((16,), cur[cc], jnp.int32)
        pltpu.sync_copy(cvec, counts.at[pl.ds(c * 16, 16)])
        nb = (cur[cc] + 2047) // 2048

        def wr_body(bb, _, cc=cc, c=c):
            pltpu.sync_copy(
                lists.at[pl.ds(cc * (CAPC + 16) + bb * 2048, 2048)],
                buckets.at[pl.ds(c * CAPC + bb * 2048, 2048)])
            return 0

        lax.fori_loop(0, nb, wr_body, 0)


def _make_edge_kernel(d_in):
    roww = QW + VW + D_M + d_in
    voff = QW
    moff = QW + VW
    xoff = QW + VW + D_M
    nxv = d_in // 16
    inv_sqrt = 1.0 / math.sqrt(float(D_A))

    @functools.partial(
        pl.kernel,
        out_type=(
            jax.ShapeDtypeStruct((NP, 16), jnp.float32),
            jax.ShapeDtypeStruct((NP, VW), jnp.float32),
            jax.ShapeDtypeStruct((NP, D_M), jnp.float32),
            jax.ShapeDtypeStruct((NP, d_in), jnp.float32),
        ),
        mesh=_MESH,
        scratch_types=[
            pltpu.VMEM((CS, QW), jnp.float32),
            pltpu.VMEM((CS, 16), jnp.float32),
            pltpu.VMEM((CS, VW), jnp.float32),
            pltpu.VMEM((CS, VW), jnp.float32),
            pltpu.VMEM((CS, D_M), jnp.float32),
            pltpu.VMEM((CS, d_in), jnp.float32),
            pltpu.VMEM((2 * GBLK,), jnp.int32),
            pltpu.VMEM((GBLK,), jnp.int32),
            pltpu.VMEM((GBLK,), jnp.int32),
            pltpu.VMEM((2 * GBLK, roww), jnp.float32),
            pltpu.VMEM((16,), jnp.int32),
            pltpu.SemaphoreType.DMA,
            pltpu.SemaphoreType.DMA,
        ],
        compiler_params=pltpu.CompilerParams(needs_layout_passes=False),
    )
    def edge_kernel(q_hbm, r_hbm, buckets, counts,
                    denom, aggv, maxm, sumx,
                    qv, accD, accA, accA2, accM, accX,
                    blk, idx0, idx1, rows, cvec, sem0, sem1):
        wid = _wid()
        zero16 = jnp.zeros((16,), jnp.float32)
        neg = jnp.full((16,), -3.0e38, jnp.float32)
        one16 = jnp.full((16,), 1.0, jnp.float32)
        low8 = lax.iota(jnp.int32, 16) < 8

        def chunk_body(cc, _):
            c = wid * CPT + cc
            base = c * CS
            pltpu.sync_copy(counts.at[pl.ds(c * 16, 16)], cvec)
            ctotal = cvec[...][0]
            pltpu.sync_copy(q_hbm.at[pl.ds(base, CS)], qv)

            def init_body(i, _):
                accD[i, :] = zero16
                for j in range(VW // 16):
                    accA[i, pl.ds(j * 16, 16)] = zero16
                    accA2[i, pl.ds(j * 16, 16)] = zero16
                for j in range(D_M // 16):
                    accM[i, pl.ds(j * 16, 16)] = neg
                for j in range(nxv):
                    accX[i, pl.ds(j * 16, 16)] = zero16
                return 0

            lax.fori_loop(0, CS, init_body, 0)

            nfull = ctotal // GBLK
            tail = ctotal - nfull * GBLK

            def fire(b, c=c):
                par = lax.rem(b, 2)
                hb = par * GBLK

                @pl.when(par == 0)
                def _():
                    pltpu.sync_copy(
                        buckets.at[pl.ds(c * CAPC + b * GBLK, GBLK)],
                        blk.at[pl.ds(0, GBLK)])
                    for jj in range(GBLK // 16):
                        sl = pl.ds(jj * 16, 16)
                        idx0[sl] = jnp.clip(
                            lax.shift_right_logical(blk[sl], 6), 0, NP - 1)
                    pltpu.async_copy(r_hbm.at[idx0],
                                     rows.at[pl.ds(0, GBLK)], sem0)

                @pl.when(par == 1)
                def _():
                    pltpu.sync_copy(
                        buckets.at[pl.ds(c * CAPC + b * GBLK, GBLK)],
                        blk.at[pl.ds(GBLK, GBLK)])
                    for jj in range(GBLK // 16):
                        idx1[pl.ds(jj * 16, 16)] = jnp.clip(
                            lax.shift_right_logical(
                                blk[pl.ds(GBLK + jj * 16, 16)], 6),
                            0, NP - 1)
                    pltpu.async_copy(r_hbm.at[idx1],
                                     rows.at[pl.ds(GBLK, GBLK)], sem1)
                del hb

            def drain(b):
                par = lax.rem(b, 2)

                @pl.when(par == 0)
                def _():
                    pltpu.make_async_copy(r_hbm.at[idx0],
                                          rows.at[pl.ds(0, GBLK)],
                                          sem0).wait()

                @pl.when(par == 1)
                def _():
                    pltpu.make_async_copy(r_hbm.at[idx1],
                                          rows.at[pl.ds(GBLK, GBLK)],
                                          sem1).wait()

            def do_edge(e, dl, aA):
                acc0 = rows[e, pl.ds(0, 16)] * qv[dl, pl.ds(0, 16)]
                acc1 = rows[e, pl.ds(16, 16)] * qv[dl, pl.ds(16, 16)]
                acc2 = rows[e, pl.ds(32, 16)] * qv[dl, pl.ds(32, 16)]
                for j in range(3, QW // 16, 3):
                    acc0 = acc0 + rows[e, pl.ds(j * 16, 16)] * qv[dl, pl.ds(j * 16, 16)]
                    acc1 = acc1 + rows[e, pl.ds(j * 16 + 16, 16)] * qv[dl, pl.ds(j * 16 + 16, 16)]
                    acc2 = acc2 + rows[e, pl.ds(j * 16 + 32, 16)] * qv[dl, pl.ds(j * 16 + 32, 16)]
                acc = acc0 + acc1 + acc2
                exv = jnp.exp((acc + lax.rev(acc, (0,))) * inv_sqrt)
                accD[dl, :] = accD[dl, :] + jnp.where(low8, exv, one16)
                for j in range(VW // 16):
                    sl = pl.ds(j * 16, 16)
                    w = jnp.full((16,), exv[j // 2], jnp.float32)
                    aA[dl, sl] = (aA[dl, sl]
                                  + rows[e, pl.ds(voff + j * 16, 16)] * w)
                for j in range(D_M // 16):
                    sl = pl.ds(j * 16, 16)
                    accM[dl, sl] = jnp.maximum(
                        accM[dl, sl], rows[e, pl.ds(moff + j * 16, 16)])
                for j in range(nxv):
                    sl = pl.ds(j * 16, 16)
                    accX[dl, sl] = (accX[dl, sl]
                                    + rows[e, pl.ds(xoff + j * 16, 16)])

            @pl.when(nfull > 0)
            def _():
                fire(jnp.int32(0))

            def blk_body(b, _):
                @pl.when(b + 1 < nfull)
                def _():
                    fire(b + 1)

                drain(b)
                hb = lax.rem(b, 2) * GBLK

                def grp_body(jj, _):
                    s0 = hb + jj * 16
                    ev = blk[pl.ds(s0, 16)]
                    dlv = ev & 63
                    for e2 in range(16):
                        do_edge(s0 + e2, dlv[e2], accA if e2 % 2 == 0 else accA2)
                    return 0

                lax.fori_loop(0, GBLK // 16, grp_body, 0)
                return 0

            lax.fori_loop(0, nfull, blk_body, 0)

            @pl.when(tail > 0)
            def _():
                pltpu.sync_copy(
                    buckets.at[pl.ds(c * CAPC + nfull * GBLK, GBLK)],
                    blk.at[pl.ds(0, GBLK)])
                for jj in range(GBLK // 16):
                    sl = pl.ds(jj * 16, 16)
                    idx0[sl] = jnp.clip(
                        lax.shift_right_logical(blk[sl], 6), 0, NP - 1)
                pltpu.async_copy(r_hbm.at[idx0],
                                 rows.at[pl.ds(0, GBLK)], sem0).wait()

                def tgrp_body(jj, _):
                    ev = blk[pl.ds(jj * 16, 16)]
                    dlv = ev & 63
                    for e2 in range(16):
                        e = jj * 16 + e2

                        @pl.when(e < tail)
                        def _(e=e, e2=e2):
                            do_edge(e, dlv[e2], accA if e2 % 2 == 0 else accA2)
                    return 0

                lax.fori_loop(0, GBLK // 16, tgrp_body, 0)

            def merge_body(i, _):
                for j in range(VW // 16):
                    sl = pl.ds(j * 16, 16)
                    accA[i, sl] = accA[i, sl] + accA2[i, sl]
                return 0

            lax.fori_loop(0, CS, merge_body, 0)

            pltpu.sync_copy(accD, denom.at[pl.ds(base, CS)])
            pltpu.sync_copy(accA, aggv.at[pl.ds(base, CS)])
            pltpu.sync_copy(accM, maxm.at[pl.ds(base, CS)])
            pltpu.sync_copy(accX, sumx.at[pl.ds(base, CS)])
            return 0

        lax.fori_loop(0, CPT, chunk_body, 0)

    return edge_kernel


_edge_kernel_128 = _make_edge_kernel(128)
_edge_kernel_256 = _make_edge_kernel(256)


def _proj_body(x_ref, wq_ref, wk_ref, wv_ref, wm_ref, q_ref, r_ref):
    xb = x_ref[...]
    q_ref[...] = jnp.dot(xb, wq_ref[...], preferred_element_type=jnp.float32)
    r_ref[:, 0:QW] = jnp.dot(xb, wk_ref[...],
                             preferred_element_type=jnp.float32)
    r_ref[:, QW:QW + VW] = jnp.dot(xb, wv_ref[...],
                                   preferred_element_type=jnp.float32)
    r_ref[:, QW + VW:QW + VW + D_M] = jnp.dot(
        xb, wm_ref[...], preferred_element_type=jnp.float32)
    r_ref[:, QW + VW + D_M:] = xb


def _proj(x, wq, wk, wv, wm):
    d = x.shape[1]
    roww = QW + VW + D_M + d
    br = 1024
    grid = (NP // br,)
    return pl.pallas_call(
        _proj_body,
        grid=grid,
        in_specs=[
            pl.BlockSpec((br, d), lambda i: (i, 0)),
            pl.BlockSpec((d, QW), lambda i: (0, 0)),
            pl.BlockSpec((d, QW), lambda i: (0, 0)),
            pl.BlockSpec((d, VW), lambda i: (0, 0)),
            pl.BlockSpec((d, D_M), lambda i: (0, 0)),
        ],
        out_specs=[
            pl.BlockSpec((br, QW), lambda i: (i, 0)),
            pl.BlockSpec((br, roww), lambda i: (i, 0)),
        ],
        out_shape=[
            jax.ShapeDtypeStruct((NP, QW), jnp.float32),
            jax.ShapeDtypeStruct((NP, roww), jnp.float32),
        ],
    )(x, wq, wk, wv, wm)


def _make_finish_body(final):
    def body(x_ref, denom_ref, aggv_ref, maxm_ref, sumx_ref,
             wgx_ref, wgm_ref, wgz_ref, wox_ref, wog_ref, o_ref):
        xb = x_ref[...]
        cnt1 = denom_ref[:, 8:9]
        has = cnt1 > 0.0
        mean = sumx_ref[...] / jnp.maximum(cnt1, 1.0)
        mm = jnp.where(has, maxm_ref[...], 0.0)
        g = jax.nn.sigmoid(
            jnp.dot(xb, wgx_ref[...], preferred_element_type=jnp.float32)
            + jnp.dot(mm, wgm_ref[...], preferred_element_type=jnp.float32)
            + jnp.dot(mean, wgz_ref[...], preferred_element_type=jnp.float32))
        expand = jnp.repeat(jnp.eye(HEADS, dtype=jnp.float32), D_V, axis=1)
        dnex = jnp.dot(denom_ref[:, 0:HEADS], expand,
                       preferred_element_type=jnp.float32)
        gex = jnp.dot(g, expand, preferred_element_type=jnp.float32)
        gated = gex * aggv_ref[...] / (dnex + 1e-16)
        out = (jnp.dot(xb, wox_ref[...], preferred_element_type=jnp.float32)
               + jnp.dot(gated, wog_ref[...],
                         preferred_element_type=jnp.float32))
        out = jnp.where(out >= 0.0, out, 0.1 * out)
        if final:
            m = jnp.max(out, axis=1, keepdims=True)
            out = out - m - jnp.log(
                jnp.sum(jnp.exp(out - m), axis=1, keepdims=True))
        o_ref[...] = out

    return body


def _finish(x, denom, aggv, maxm, sumx, wg, wo, final):
    d = x.shape[1]
    do = wo.shape[1]
    wgx = wg[:d]
    wgm = wg[d:d + D_M]
    wgz = wg[d + D_M:]
    wox = wo[:d]
    wog = wo[d:]
    br = 1024
    grid = (NP // br,)
    return pl.pallas_call(
        _make_finish_body(final),
        grid=grid,
        in_specs=[
            pl.BlockSpec((br, d), lambda i: (i, 0)),
            pl.BlockSpec((br, 16), lambda i: (i, 0)),
            pl.BlockSpec((br, VW), lambda i: (i, 0)),
            pl.BlockSpec((br, D_M), lambda i: (i, 0)),
            pl.BlockSpec((br, d), lambda i: (i, 0)),
            pl.BlockSpec((d, HEADS), lambda i: (0, 0)),
            pl.BlockSpec((D_M, HEADS), lambda i: (0, 0)),
            pl.BlockSpec((d, HEADS), lambda i: (0, 0)),
            pl.BlockSpec((d, do), lambda i: (0, 0)),
            pl.BlockSpec((VW, do), lambda i: (0, 0)),
        ],
        out_specs=pl.BlockSpec((br, do), lambda i: (i, 0)),
        out_shape=jax.ShapeDtypeStruct((NP, do), jnp.float32),
    )(x, denom, aggv, maxm, sumx, wgx, wgm, wgz, wox, wog)


def _layer(x, buckets, counts, Wq, Wk, Wv, Wm, Wg, Wo, edge_kernel, final):
    d = x.shape[1]
    perm = []
    for j in range(QW // 16):
        for lane in range(16):
            h = lane if lane < 8 else 15 - lane
            a = 2 * j if lane < 8 else 2 * j + 1
            perm.append(h * D_A + a)
    perm = jnp.asarray(perm, jnp.int32)
    wq_p = Wq[:, perm]
    wk_p = Wk[:, perm]
    q, r = _proj(x, wq_p, wk_p, Wv, Wm)
    denom, aggv, maxm, sumx = edge_kernel(q, r, buckets, counts)
    return _finish(x, denom, aggv, maxm, sumx, Wg, Wo, final)


def kernel(x, edge_index, Wq1, Wk1, Wv1, Wm1, Wg1, Wo1,
           Wq2, Wk2, Wv2, Wm2, Wg2, Wo2):
    src = edge_index[0]
    dst = edge_index[1]
    buckets, counts = _bucket_kernel(src, dst)
    xp = jnp.pad(x, ((0, NP - N), (0, 0)))
    h = _layer(xp, buckets, counts, Wq1, Wk1, Wv1, Wm1, Wg1, Wo1,
               _edge_kernel_128, final=False)
    out = _layer(h, buckets, counts, Wq2, Wk2, Wv2, Wm2, Wg2, Wo2,
                 _edge_kernel_256, final=True)
    return out[:N]


# speedup vs baseline: 1.3260x; 1.1160x over previous
import functools
import math

import jax
import jax.numpy as jnp
from jax import lax
from jax.experimental import pallas as pl
from jax.experimental.pallas import tpu as pltpu
from jax.experimental.pallas import tpu_sc as plsc

N = 10000
E = 320000
HEADS = 8
D_A = 24
D_V = 32
D_M = 64

NP = 10240
CS = 64
NCHUNKS = NP // CS
NTILES = 32
CPT = NCHUNKS // NTILES
CAPC = 16384
EBLK = 2000
GBLK = 16
QW = HEADS * D_A
VW = HEADS * D_V

_MESH = plsc.VectorSubcoreMesh(core_axis_name="c", subcore_axis_name="s")


def _wid():
    return lax.axis_index("s") * 2 + lax.axis_index("c")


@functools.partial(
    pl.kernel,
    out_type=(
        jax.ShapeDtypeStruct((NCHUNKS * CAPC,), jnp.int32),
        jax.ShapeDtypeStruct((NCHUNKS * 16,), jnp.int32),
    ),
    mesh=_MESH,
    scratch_types=[
        pltpu.VMEM((EBLK,), jnp.int32),
        pltpu.VMEM((EBLK,), jnp.int32),
        pltpu.VMEM((CPT * (CAPC + 16),), jnp.int32),
        pltpu.VMEM((16,), jnp.int32),
    ],
    compiler_params=pltpu.CompilerParams(needs_layout_passes=False),
)
def _bucket_kernel(src_hbm, dst_hbm, buckets, counts, sblk, dblk, lists, cvec):
    wid = _wid()
    base_node = wid * (CPT * CS)

    def blk_body(b, cur):
        pltpu.sync_copy(src_hbm.at[pl.ds(b * EBLK, EBLK)], sblk)
        pltpu.sync_copy(dst_hbm.at[pl.ds(b * EBLK, EBLK)], dblk)

        def vec_body(j, cur):
            sv = sblk[pl.ds(j * 16, 16)]
            dv = dblk[pl.ds(j * 16, 16)]
            dlt = dv - base_node
            new = []
            for cc in range(CPT):
                lo = cc * CS
                mask = (dlt >= lo) & (dlt < lo + CS)
                packed = (sv << 6) | (dlt - lo)
                mi = jnp.where(mask, jnp.full((16,), 1, jnp.int32),
                               jnp.full((16,), 0, jnp.int32))
                incl = plsc.cumsum(mi)
                base_pos = jnp.full((16,), cc * (CAPC + 16) + cur[cc],
                                    jnp.int32)
                pos = base_pos + (incl - mi)
                plsc.store_scatter(lists, [pos], packed, mask=mask)
                new.append(cur[cc] + incl[15])
            return tuple(new)

        return lax.fori_loop(0, EBLK // 16, vec_body, cur)

    cur = lax.fori_loop(0, E // EBLK, blk_body,
                        tuple(jnp.int32(0) for _ in range(CPT)))

    for cc in range(CPT):
        c = wid * CPT + cc
        cvec[...] = jnp.full((16,), cur[cc], jnp.int32)
        pltpu.sync_copy(cvec, counts.at[pl.ds(c * 16, 16)])
        nb = (cur[cc] + 2047) // 2048

        def wr_body(bb, _, cc=cc, c=c):
            pltpu.sync_copy(
                lists.at[pl.ds(cc * (CAPC + 16) + bb * 2048, 2048)],
                buckets.at[pl.ds(c * CAPC + bb * 2048, 2048)])
            return 0

        lax.fori_loop(0, nb, wr_body, 0)


def _make_edge_kernel(d_in):
    roww = QW + VW + D_M + d_in
    voff = QW
    moff = QW + VW
    xoff = QW + VW + D_M
    nxv = d_in // 16
    inv_sqrt = 1.0 / math.sqrt(float(D_A))

    @functools.partial(
        pl.kernel,
        out_type=(
            jax.ShapeDtypeStruct((NP, 16), jnp.float32),
            jax.ShapeDtypeStruct((NP, VW), jnp.float32),
            jax.ShapeDtypeStruct((NP, D_M), jnp.float32),
            jax.ShapeDtypeStruct((NP, d_in), jnp.float32),
        ),
        mesh=_MESH,
        scratch_types=[
            pltpu.VMEM((CS, QW), jnp.float32),
            pltpu.VMEM((CS, 16), jnp.float32),
            pltpu.VMEM((CS, VW), jnp.float32),
            pltpu.VMEM((CS, VW), jnp.float32),
            pltpu.VMEM((CS, D_M), jnp.float32),
            pltpu.VMEM((CS, d_in), jnp.float32),
            pltpu.VMEM((CAPC,), jnp.int32),
            pltpu.VMEM((GBLK,), jnp.int32),
            pltpu.VMEM((GBLK,), jnp.int32),
            pltpu.VMEM((2 * GBLK, roww), jnp.float32),
            pltpu.VMEM((16,), jnp.int32),
            pltpu.SemaphoreType.DMA,
            pltpu.SemaphoreType.DMA,
        ],
        compiler_params=pltpu.CompilerParams(needs_layout_passes=False),
    )
    def edge_kernel(q_hbm, r_hbm, buckets, counts,
                    denom, aggv, maxm, sumx,
                    qv, accD, accA, accA2, accM, accX,
                    ebuf, idx0, idx1, rows, cvec, sem0, sem1):
        wid = _wid()
        zero16 = jnp.zeros((16,), jnp.float32)
        neg = jnp.full((16,), -3.0e38, jnp.float32)
        one16 = jnp.full((16,), 1.0, jnp.float32)
        low8 = lax.iota(jnp.int32, 16) < 8

        def chunk_body(cc, _):
            c = wid * CPT + cc
            base = c * CS
            pltpu.sync_copy(counts.at[pl.ds(c * 16, 16)], cvec)
            ctotal = cvec[...][0]
            pltpu.sync_copy(q_hbm.at[pl.ds(base, CS)], qv)

            def init_body(i, _):
                accD[i, :] = zero16
                for j in range(VW // 16):
                    accA[i, pl.ds(j * 16, 16)] = zero16
                    accA2[i, pl.ds(j * 16, 16)] = zero16
                for j in range(D_M // 16):
                    accM[i, pl.ds(j * 16, 16)] = neg
                for j in range(nxv):
                    accX[i, pl.ds(j * 16, 16)] = zero16
                return 0

            lax.fori_loop(0, CS, init_body, 0)

            nfull = ctotal // GBLK
            tail = ctotal - nfull * GBLK
            nbig = (ctotal + 2047) // 2048

            def ld_body(i, _, c=c):
                pltpu.sync_copy(
                    buckets.at[pl.ds(c * CAPC + i * 2048, 2048)],
                    ebuf.at[pl.ds(i * 2048, 2048)])
                return 0

            lax.fori_loop(0, nbig, ld_body, 0)

            def fire(b):
                par = lax.rem(b, 2)

                @pl.when(par == 0)
                def _():
                    for jj in range(GBLK // 16):
                        idx0[pl.ds(jj * 16, 16)] = jnp.clip(
                            lax.shift_right_logical(
                                ebuf[pl.ds(b * GBLK + jj * 16, 16)], 6),
                            0, NP - 1)
                    pltpu.async_copy(r_hbm.at[idx0],
                                     rows.at[pl.ds(0, GBLK)], sem0)

                @pl.when(par == 1)
                def _():
                    for jj in range(GBLK // 16):
                        idx1[pl.ds(jj * 16, 16)] = jnp.clip(
                            lax.shift_right_logical(
                                ebuf[pl.ds(b * GBLK + jj * 16, 16)], 6),
                            0, NP - 1)
                    pltpu.async_copy(r_hbm.at[idx1],
                                     rows.at[pl.ds(GBLK, GBLK)], sem1)

            def drain(b):
                par = lax.rem(b, 2)

                @pl.when(par == 0)
                def _():
                    pltpu.make_async_copy(r_hbm.at[idx0],
                                          rows.at[pl.ds(0, GBLK)],
                                          sem0).wait()

                @pl.when(par == 1)
                def _():
                    pltpu.make_async_copy(r_hbm.at[idx1],
                                          rows.at[pl.ds(GBLK, GBLK)],
                                          sem1).wait()

            def do_edge(e, dl, aA):
                acc0 = rows[e, pl.ds(0, 16)] * qv[dl, pl.ds(0, 16)]
                acc1 = rows[e, pl.ds(16, 16)] * qv[dl, pl.ds(16, 16)]
                acc2 = rows[e, pl.ds(32, 16)] * qv[dl, pl.ds(32, 16)]
                for j in range(3, QW // 16, 3):
                    acc0 = acc0 + rows[e, pl.ds(j * 16, 16)] * qv[dl, pl.ds(j * 16, 16)]
                    acc1 = acc1 + rows[e, pl.ds(j * 16 + 16, 16)] * qv[dl, pl.ds(j * 16 + 16, 16)]
                    acc2 = acc2 + rows[e, pl.ds(j * 16 + 32, 16)] * qv[dl, pl.ds(j * 16 + 32, 16)]
                acc = acc0 + acc1 + acc2
                exv = jnp.exp((acc + lax.rev(acc, (0,))) * inv_sqrt)
                accD[dl, :] = accD[dl, :] + jnp.where(low8, exv, one16)
                for j in range(VW // 16):
                    sl = pl.ds(j * 16, 16)
                    w = jnp.full((16,), exv[j // 2], jnp.float32)
                    aA[dl, sl] = (aA[dl, sl]
                                  + rows[e, pl.ds(voff + j * 16, 16)] * w)
                for j in range(D_M // 16):
                    sl = pl.ds(j * 16, 16)
                    accM[dl, sl] = jnp.maximum(
                        accM[dl, sl], rows[e, pl.ds(moff + j * 16, 16)])
                for j in range(nxv):
                    sl = pl.ds(j * 16, 16)
                    accX[dl, sl] = (accX[dl, sl]
                                    + rows[e, pl.ds(xoff + j * 16, 16)])

            @pl.when(nfull > 0)
            def _():
                fire(jnp.int32(0))

            def blk_body(b, _):
                @pl.when(b + 1 < nfull)
                def _():
                    fire(b + 1)

                drain(b)
                hb = lax.rem(b, 2) * GBLK

                def grp_body(jj, _):
                    s0 = hb + jj * 16
                    ev = ebuf[pl.ds(b * GBLK + jj * 16, 16)]
                    dlv = ev & 63
                    for e2 in range(16):
                        do_edge(s0 + e2, dlv[e2], accA if e2 % 2 == 0 else accA2)
                    return 0

                lax.fori_loop(0, GBLK // 16, grp_body, 0)
                return 0

            lax.fori_loop(0, nfull, blk_body, 0)

            @pl.when(tail > 0)
            def _():
                for jj in range(GBLK // 16):
                    idx0[pl.ds(jj * 16, 16)] = jnp.clip(
                        lax.shift_right_logical(
                            ebuf[pl.ds(nfull * GBLK + jj * 16, 16)], 6),
                        0, NP - 1)
                pltpu.async_copy(r_hbm.at[idx0],
                                 rows.at[pl.ds(0, GBLK)], sem0).wait()

                def tgrp_body(jj, _):
                    ev = ebuf[pl.ds(nfull * GBLK + jj * 16, 16)]
                    dlv = ev & 63
                    for e2 in range(16):
                        e = jj * 16 + e2

                        @pl.when(e < tail)
                        def _(e=e, e2=e2):
                            do_edge(e, dlv[e2], accA if e2 % 2 == 0 else accA2)
                    return 0

                lax.fori_loop(0, GBLK // 16, tgrp_body, 0)

            def merge_body(i, _):
                for j in range(VW // 16):
                    sl = pl.ds(j * 16, 16)
                    accA[i, sl] = accA[i, sl] + accA2[i, sl]
                return 0

            lax.fori_loop(0, CS, merge_body, 0)

            pltpu.sync_copy(accD, denom.at[pl.ds(base, CS)])
            pltpu.sync_copy(accA, aggv.at[pl.ds(base, CS)])
            pltpu.sync_copy(accM, maxm.at[pl.ds(base, CS)])
            pltpu.sync_copy(accX, sumx.at[pl.ds(base, CS)])
            return 0

        lax.fori_loop(0, CPT, chunk_body, 0)

    return edge_kernel


_edge_kernel_128 = _make_edge_kernel(128)
_edge_kernel_256 = _make_edge_kernel(256)


def _proj_body(x_ref, wq_ref, wk_ref, wv_ref, wm_ref, q_ref, r_ref):
    xb = x_ref[...]
    q_ref[...] = jnp.dot(xb, wq_ref[...], preferred_element_type=jnp.float32)
    r_ref[:, 0:QW] = jnp.dot(xb, wk_ref[...],
                             preferred_element_type=jnp.float32)
    r_ref[:, QW:QW + VW] = jnp.dot(xb, wv_ref[...],
                                   preferred_element_type=jnp.float32)
    r_ref[:, QW + VW:QW + VW + D_M] = jnp.dot(
        xb, wm_ref[...], preferred_element_type=jnp.float32)
    r_ref[:, QW + VW + D_M:] = xb


def _proj(x, wq, wk, wv, wm):
    d = x.shape[1]
    roww = QW + VW + D_M + d
    br = 1024
    grid = (NP // br,)
    return pl.pallas_call(
        _proj_body,
        grid=grid,
        in_specs=[
            pl.BlockSpec((br, d), lambda i: (i, 0)),
            pl.BlockSpec((d, QW), lambda i: (0, 0)),
            pl.BlockSpec((d, QW), lambda i: (0, 0)),
            pl.BlockSpec((d, VW), lambda i: (0, 0)),
            pl.BlockSpec((d, D_M), lambda i: (0, 0)),
        ],
        out_specs=[
            pl.BlockSpec((br, QW), lambda i: (i, 0)),
            pl.BlockSpec((br, roww), lambda i: (i, 0)),
        ],
        out_shape=[
            jax.ShapeDtypeStruct((NP, QW), jnp.float32),
            jax.ShapeDtypeStruct((NP, roww), jnp.float32),
        ],
    )(x, wq, wk, wv, wm)


def _make_finish_body(final):
    def body(x_ref, denom_ref, aggv_ref, maxm_ref, sumx_ref,
             wgx_ref, wgm_ref, wgz_ref, wox_ref, wog_ref, o_ref):
        xb = x_ref[...]
        cnt1 = denom_ref[:, 8:9]
        has = cnt1 > 0.0
        mean = sumx_ref[...] / jnp.maximum(cnt1, 1.0)
        mm = jnp.where(has, maxm_ref[...], 0.0)
        g = jax.nn.sigmoid(
            jnp.dot(xb, wgx_ref[...], preferred_element_type=jnp.float32)
            + jnp.dot(mm, wgm_ref[...], preferred_element_type=jnp.float32)
            + jnp.dot(mean, wgz_ref[...], preferred_element_type=jnp.float32))
        expand = jnp.repeat(jnp.eye(HEADS, dtype=jnp.float32), D_V, axis=1)
        dnex = jnp.dot(denom_ref[:, 0:HEADS], expand,
                       preferred_element_type=jnp.float32)
        gex = jnp.dot(g, expand, preferred_element_type=jnp.float32)
        gated = gex * aggv_ref[...] / (dnex + 1e-16)
        out = (jnp.dot(xb, wox_ref[...], preferred_element_type=jnp.float32)
               + jnp.dot(gated, wog_ref[...],
                         preferred_element_type=jnp.float32))
        out = jnp.where(out >= 0.0, out, 0.1 * out)
        if final:
            m = jnp.max(out, axis=1, keepdims=True)
            out = out - m - jnp.log(
                jnp.sum(jnp.exp(out - m), axis=1, keepdims=True))
        o_ref[...] = out

    return body


def _finish(x, denom, aggv, maxm, sumx, wg, wo, final):
    d = x.shape[1]
    do = wo.shape[1]
    wgx = wg[:d]
    wgm = wg[d:d + D_M]
    wgz = wg[d + D_M:]
    wox = wo[:d]
    wog = wo[d:]
    br = 1024
    grid = (NP // br,)
    return pl.pallas_call(
        _make_finish_body(final),
        grid=grid,
        in_specs=[
            pl.BlockSpec((br, d), lambda i: (i, 0)),
            pl.BlockSpec((br, 16), lambda i: (i, 0)),
            pl.BlockSpec((br, VW), lambda i: (i, 0)),
            pl.BlockSpec((br, D_M), lambda i: (i, 0)),
            pl.BlockSpec((br, d), lambda i: (i, 0)),
            pl.BlockSpec((d, HEADS), lambda i: (0, 0)),
            pl.BlockSpec((D_M, HEADS), lambda i: (0, 0)),
            pl.BlockSpec((d, HEADS), lambda i: (0, 0)),
            pl.BlockSpec((d, do), lambda i: (0, 0)),
            pl.BlockSpec((VW, do), lambda i: (0, 0)),
        ],
        out_specs=pl.BlockSpec((br, do), lambda i: (i, 0)),
        out_shape=jax.ShapeDtypeStruct((NP, do), jnp.float32),
    )(x, denom, aggv, maxm, sumx, wgx, wgm, wgz, wox, wog)


def _layer(x, buckets, counts, Wq, Wk, Wv, Wm, Wg, Wo, edge_kernel, final):
    d = x.shape[1]
    perm = []
    for j in range(QW // 16):
        for lane in range(16):
            h = lane if lane < 8 else 15 - lane
            a = 2 * j if lane < 8 else 2 * j + 1
            perm.append(h * D_A + a)
    perm = jnp.asarray(perm, jnp.int32)
    wq_p = Wq[:, perm]
    wk_p = Wk[:, perm]
    q, r = _proj(x, wq_p, wk_p, Wv, Wm)
    denom, aggv, maxm, sumx = edge_kernel(q, r, buckets, counts)
    return _finish(x, denom, aggv, maxm, sumx, Wg, Wo, final)


def kernel(x, edge_index, Wq1, Wk1, Wv1, Wm1, Wg1, Wo1,
           Wq2, Wk2, Wv2, Wm2, Wg2, Wo2):
    src = edge_index[0]
    dst = edge_index[1]
    buckets, counts = _bucket_kernel(src, dst)
    xp = jnp.pad(x, ((0, NP - N), (0, 0)))
    h = _layer(xp, buckets, counts, Wq1, Wk1, Wv1, Wm1, Wg1, Wo1,
               _edge_kernel_128, final=False)
    out = _layer(h, buckets, counts, Wq2, Wk2, Wv2, Wm2, Wg2, Wo2,
                 _edge_kernel_256, final=True)
    return out[:N]
